# Initial kernel scaffold; baseline (speedup 1.0000x reference)
import jax
import jax.numpy as jnp
from jax.experimental import pallas as pl


def kernel(confidence, bboxes):
    raise NotImplementedError("write your pallas kernel here")


# speedup vs baseline: 20.5569x; 20.5569x over previous
import functools

import jax
import jax.numpy as jnp
from jax.experimental import pallas as pl
from jax.experimental.pallas import tpu as pltpu

_TOP_K_RATIO = 0.5
_IOU_THR = 0.5
_OUT_P = 200
_OUT_PAD = 256


def _nms_body(K, Kp, R, C, y1_ref, x1_ref, y2_ref, x2_ref, data_ref,
              out_ref, keep_ref, sel_ref):
    y1 = y1_ref[0]
    x1 = x1_ref[0]
    y2 = y2_ref[0]
    x2 = x2_ref[0]
    area = (y2 - y1) * (x2 - x1)

    ar = (jax.lax.broadcasted_iota(jnp.int32, (R, C), 0) * C
          + jax.lax.broadcasted_iota(jnp.int32, (R, C), 1))
    keep_ref[...] = (ar < K).astype(jnp.int32)
    j256 = jax.lax.broadcasted_iota(jnp.int32, (1, _OUT_PAD), 1)
    sel_ref[...] = jnp.zeros((1, _OUT_PAD), jnp.int32)

    def cond(state):
        i, count = state
        return (i < K) & (count < _OUT_P)

    def body(state):
        i, count = state
        keep = keep_ref[...]
        mask = (ar == i)
        ki = jnp.sum(jnp.where(mask, keep, 0))

        @pl.when(ki > 0)
        def _():
            maskf = mask.astype(jnp.float32)
            y1i = jnp.sum(y1 * maskf)
            x1i = jnp.sum(x1 * maskf)
            y2i = jnp.sum(y2 * maskf)
            x2i = jnp.sum(x2 * maskf)
            ai = (y2i - y1i) * (x2i - x1i)
            ih = jnp.maximum(jnp.minimum(y2i, y2) - jnp.maximum(y1i, y1), 0.0)
            iw = jnp.maximum(jnp.minimum(x2i, x2) - jnp.maximum(x1i, x1), 0.0)
            inter = ih * iw
            union = ai + area - inter
            iou = jnp.where(union > 0.0, inter / union, 0.0)
            sup = (iou > _IOU_THR) & (ar > i)
            keep_ref[...] = jnp.where(sup, 0, keep)
            sel_ref[...] = jnp.where(j256 == count, i, sel_ref[...])

        return i + 1, count + ki

    _, count = jax.lax.while_loop(cond, body, (jnp.int32(0), jnp.int32(0)))

    selv = sel_ref[...]
    last_idx = jnp.sum(jnp.where(j256 == count - 1, selv, 0))
    start = jnp.minimum(K - (_OUT_P - count), last_idx + 1)
    final = jnp.where(j256 < count, selv, start + (j256 - count))
    final = jnp.clip(final, 0, K - 1)

    oh = (jax.lax.broadcasted_iota(jnp.int32, (Kp, _OUT_PAD), 0)
          == final).astype(jnp.float32)
    lhs = data_ref[0]
    out_ref[...] = jnp.dot(lhs, oh,
                           preferred_element_type=jnp.float32)[None]


def kernel(confidence, bboxes):
    B, A = confidence.shape
    K = int(A * _TOP_K_RATIO)
    C = 128
    Kp = ((K + C - 1) // C) * C
    R = Kp // C

    conf_k, idxs = jax.lax.top_k(confidence, K)
    box_idx = jnp.broadcast_to(idxs[:, :, None], (B, K, 4))
    boxes_k = jnp.take_along_axis(bboxes, box_idx, axis=1)

    pad = Kp - K
    confp = jnp.pad(conf_k, ((0, 0), (0, pad)))
    comps = [jnp.pad(boxes_k[:, :, d], ((0, 0), (0, pad))) for d in range(4)]
    data = jnp.stack([confp] + comps, axis=1)
    grids = [c.reshape(B, R, C) for c in comps]

    body = functools.partial(_nms_body, K, Kp, R, C)
    out = pl.pallas_call(
        body,
        grid=(B,),
        in_specs=[pl.BlockSpec((1, R, C), lambda b: (b, 0, 0))] * 4
        + [pl.BlockSpec((1, 5, Kp), lambda b: (b, 0, 0))],
        out_specs=pl.BlockSpec((1, 5, _OUT_PAD), lambda b: (b, 0, 0)),
        out_shape=jax.ShapeDtypeStruct((B, 5, _OUT_PAD), jnp.float32),
        scratch_shapes=[
            pltpu.VMEM((R, C), jnp.int32),
            pltpu.VMEM((1, _OUT_PAD), jnp.int32),
        ],
    )(*grids, data)

    conf_out = out[:, 0, :_OUT_P]
    boxes_out = jnp.moveaxis(out[:, 1:5, :_OUT_P], 1, 2)
    return conf_out, boxes_out


# speedup vs baseline: 20.5621x; 1.0003x over previous
import functools

import jax
import jax.numpy as jnp
from jax.experimental import pallas as pl
from jax.experimental.pallas import tpu as pltpu

_TOP_K_RATIO = 0.5
_IOU_THR = 0.5
_OUT_P = 200
_OUT_PAD = 256


def _nms_body(K, Kp, R, C, y1_ref, x1_ref, y2_ref, x2_ref, data_ref,
              out_ref, keep_ref, sel_ref):
    y1 = y1_ref[0]
    x1 = x1_ref[0]
    y2 = y2_ref[0]
    x2 = x2_ref[0]
    area = (y2 - y1) * (x2 - x1)

    ar = (jax.lax.broadcasted_iota(jnp.int32, (R, C), 0) * C
          + jax.lax.broadcasted_iota(jnp.int32, (R, C), 1))
    keep_ref[...] = (ar < K).astype(jnp.int32)
    j256 = jax.lax.broadcasted_iota(jnp.int32, (1, _OUT_PAD), 1)
    sel_ref[...] = jnp.zeros((1, _OUT_PAD), jnp.int32)

    def cond(state):
        i, count = state
        return (i < K) & (count < _OUT_P)

    def body(state):
        i, count = state
        keep = keep_ref[...]
        mask = (ar == i)
        ki = jnp.sum(jnp.where(mask, keep, 0))

        @pl.when(ki > 0)
        def _():
            maskf = mask.astype(jnp.float32)
            y1i = jnp.sum(y1 * maskf)
            x1i = jnp.sum(x1 * maskf)
            y2i = jnp.sum(y2 * maskf)
            x2i = jnp.sum(x2 * maskf)
            ai = (y2i - y1i) * (x2i - x1i)
            ih = jnp.maximum(jnp.minimum(y2i, y2) - jnp.maximum(y1i, y1), 0.0)
            iw = jnp.maximum(jnp.minimum(x2i, x2) - jnp.maximum(x1i, x1), 0.0)
            inter = ih * iw
            union = ai + area - inter
            iou = jnp.where(union > 0.0, inter / union, 0.0)
            sup = (iou > _IOU_THR) & (ar > i)
            keep_ref[...] = jnp.where(sup, 0, keep)
            sel_ref[...] = jnp.where(j256 == count, i, sel_ref[...])

        return i + 1, count + ki

    _, count = jax.lax.while_loop(cond, body, (jnp.int32(0), jnp.int32(0)))

    selv = sel_ref[...]
    last_idx = jnp.sum(jnp.where(j256 == count - 1, selv, 0))
    start = jnp.minimum(K - (_OUT_P - count), last_idx + 1)
    final = jnp.where(j256 < count, selv, start + (j256 - count))
    final = jnp.clip(final, 0, K - 1)

    oh = (jax.lax.broadcasted_iota(jnp.int32, (Kp, _OUT_PAD), 0)
          == final).astype(jnp.float32)
    lhs = data_ref[0]
    out_ref[...] = jnp.dot(lhs, oh,
                           preferred_element_type=jnp.float32)[None]


def kernel(confidence, bboxes):
    B, A = confidence.shape
    K = int(A * _TOP_K_RATIO)
    C = 128
    Kp = ((K + C - 1) // C) * C
    R = Kp // C

    conf_k, idxs = jax.lax.top_k(confidence, K)
    box_idx = jnp.broadcast_to(idxs[:, :, None], (B, K, 4))
    boxes_k = jnp.take_along_axis(bboxes, box_idx, axis=1)

    pad = Kp - K
    confp = jnp.pad(conf_k, ((0, 0), (0, pad)))
    comps = [jnp.pad(boxes_k[:, :, d], ((0, 0), (0, pad))) for d in range(4)]
    data = jnp.stack([confp] + comps, axis=1)
    grids = [c.reshape(B, R, C) for c in comps]

    body = functools.partial(_nms_body, K, Kp, R, C)
    out = pl.pallas_call(
        body,
        grid=(B,),
        in_specs=[pl.BlockSpec((1, R, C), lambda b: (b, 0, 0))] * 4
        + [pl.BlockSpec((1, 5, Kp), lambda b: (b, 0, 0))],
        out_specs=pl.BlockSpec((1, 5, _OUT_PAD), lambda b: (b, 0, 0)),
        out_shape=jax.ShapeDtypeStruct((B, 5, _OUT_PAD), jnp.float32),
        scratch_shapes=[
            pltpu.VMEM((R, C), jnp.int32),
            pltpu.VMEM((1, _OUT_PAD), jnp.int32),
        ],
        compiler_params=pltpu.CompilerParams(
            dimension_semantics=("parallel",)),
    )(*grids, data)

    conf_out = out[:, 0, :_OUT_P]
    boxes_out = jnp.moveaxis(out[:, 1:5, :_OUT_P], 1, 2)
    return conf_out, boxes_out


# speedup vs baseline: 31.8445x; 1.5487x over previous
import functools

import jax
import jax.numpy as jnp
from jax.experimental import pallas as pl
from jax.experimental.pallas import tpu as pltpu

_TOP_K_RATIO = 0.5
_IOU_THR = 0.5
_OUT_P = 200
_OUT_PAD = 256


def _nms_body(K, Kp, R, C, y1_ref, x1_ref, y2_ref, x2_ref, data_ref,
              y1s_ref, x1s_ref, y2s_ref, x2s_ref,
              out_ref, keep_ref, sel_ref):
    y1 = y1_ref[0]
    x1 = x1_ref[0]
    y2 = y2_ref[0]
    x2 = x2_ref[0]
    area = (y2 - y1) * (x2 - x1)

    ar = (jax.lax.broadcasted_iota(jnp.int32, (R, C), 0) * C
          + jax.lax.broadcasted_iota(jnp.int32, (R, C), 1))
    keep_ref[...] = (ar < K).astype(jnp.int32)
    j256 = jax.lax.broadcasted_iota(jnp.int32, (1, _OUT_PAD), 1)
    sel_ref[...] = jnp.zeros((1, _OUT_PAD), jnp.int32)

    def cond(state):
        i, count = state
        return (i < K) & (count < _OUT_P)

    def body(state):
        i, count = state
        keep = keep_ref[...]
        mask = (ar == i)
        ki = jnp.sum(jnp.where(mask, keep, 0))

        @pl.when(ki > 0)
        def _():
            y1i = y1s_ref[0, 0, i]
            x1i = x1s_ref[0, 0, i]
            y2i = y2s_ref[0, 0, i]
            x2i = x2s_ref[0, 0, i]
            ai = (y2i - y1i) * (x2i - x1i)
            ih = jnp.maximum(jnp.minimum(y2i, y2) - jnp.maximum(y1i, y1), 0.0)
            iw = jnp.maximum(jnp.minimum(x2i, x2) - jnp.maximum(x1i, x1), 0.0)
            inter = ih * iw
            union = ai + area - inter
            iou = jnp.where(union > 0.0, inter / union, 0.0)
            sup = (iou > _IOU_THR) & (ar > i)
            keep_ref[...] = jnp.where(sup, 0, keep)
            sel_ref[...] = jnp.where(j256 == count, i, sel_ref[...])

        return i + 1, count + ki

    _, count = jax.lax.while_loop(cond, body, (jnp.int32(0), jnp.int32(0)))

    selv = sel_ref[...]
    last_idx = jnp.sum(jnp.where(j256 == count - 1, selv, 0))
    start = jnp.minimum(K - (_OUT_P - count), last_idx + 1)
    final = jnp.where(j256 < count, selv, start + (j256 - count))
    final = jnp.clip(final, 0, K - 1)

    oh = (jax.lax.broadcasted_iota(jnp.int32, (Kp, _OUT_PAD), 0)
          == final).astype(jnp.float32)
    lhs = data_ref[0]
    out_ref[...] = jnp.dot(lhs, oh,
                           preferred_element_type=jnp.float32)[None]


def kernel(confidence, bboxes):
    B, A = confidence.shape
    K = int(A * _TOP_K_RATIO)
    C = 128
    Kp = ((K + C - 1) // C) * C
    R = Kp // C

    conf_k, idxs = jax.lax.top_k(confidence, K)
    box_idx = jnp.broadcast_to(idxs[:, :, None], (B, K, 4))
    boxes_k = jnp.take_along_axis(bboxes, box_idx, axis=1)

    pad = Kp - K
    confp = jnp.pad(conf_k, ((0, 0), (0, pad)))
    comps = [jnp.pad(boxes_k[:, :, d], ((0, 0), (0, pad))) for d in range(4)]
    data = jnp.stack([confp] + comps, axis=1)
    grids = [c.reshape(B, R, C) for c in comps]

    body = functools.partial(_nms_body, K, Kp, R, C)
    out = pl.pallas_call(
        body,
        grid=(B,),
        in_specs=[pl.BlockSpec((1, R, C), lambda b: (b, 0, 0))] * 4
        + [pl.BlockSpec((1, 5, Kp), lambda b: (b, 0, 0))]
        + [pl.BlockSpec((1, 1, Kp), lambda b: (b, 0, 0),
                        memory_space=pltpu.SMEM)] * 4,
        out_specs=pl.BlockSpec((1, 5, _OUT_PAD), lambda b: (b, 0, 0)),
        out_shape=jax.ShapeDtypeStruct((B, 5, _OUT_PAD), jnp.float32),
        scratch_shapes=[
            pltpu.VMEM((R, C), jnp.int32),
            pltpu.VMEM((1, _OUT_PAD), jnp.int32),
        ],
        compiler_params=pltpu.CompilerParams(
            dimension_semantics=("parallel",)),
    )(*grids, data, *[c.reshape(B, 1, Kp) for c in comps])

    conf_out = out[:, 0, :_OUT_P]
    boxes_out = jnp.moveaxis(out[:, 1:5, :_OUT_P], 1, 2)
    return conf_out, boxes_out
